# baseline (speedup 1.0000x reference)
import functools

import jax
import jax.numpy as jnp
from jax import lax
from jax.experimental import pallas as pl
from jax.experimental.pallas import tpu as pltpu
from jax.experimental.pallas import tpu_sc as plsc

_K = 32
_L = 16
_NW = 32
_NT = 128
_TCH = 5


def _sc_body(N, T, params_hbm, ch_hbm, rw_hbm, out_hbm,
             params_v, ch_v, rw_v, stage_a, stage_b, vtab, sem_a, sem_b):
    wid = lax.axis_index("s") * 2 + lax.axis_index("c")
    n0 = wid * _NT

    pltpu.sync_copy(params_hbm, params_v)
    iv = params_v[pl.ds(0, _L)]
    ap = params_v[pl.ds(_L, _L)]
    am = params_v[pl.ds(2 * _L, _L)]
    iota = lax.iota(jnp.int32, _L)
    rows_s = [iota + s * _L for s in range(_NT // _L)]

    pltpu.sync_copy(ch_hbm.at[pl.ds(n0, _NT), :], ch_v.at[:, pl.ds(0, T)])
    pltpu.sync_copy(rw_hbm.at[pl.ds(n0, _NT), :], rw_v.at[:, pl.ds(0, T)])

    for k in range(_K):
        for j in range(_NT // _L):
            vtab[pl.ds(k * _NT + j * _L, _L)] = iv

    slab = _TCH * _K * _NT
    tslab = _K * (N // _NT) * _NT

    def run_chunk(c, stage_v, sem, first):
        @pl.when(jnp.logical_not(first))
        def _():
            pltpu.make_async_copy(stage_v, out_hbm.at[pl.ds(0, slab)],
                                  sem).wait()

        def step(tl, carry):
            t_vec = jnp.full((_L,), c * _TCH + tl, jnp.int32)
            sbase = tl * (_K * _NT)

            def copyk(k, cc):
                sb = sbase + k * _NT
                vb = k * _NT
                for j in range(_NT // _L):
                    stage_v[pl.ds(sb + j * _L, _L)] = (
                        vtab[pl.ds(vb + j * _L, _L)])
                return cc

            lax.fori_loop(0, _K, copyk, 0)
            for s in range(_NT // _L):
                ch = plsc.load_gather(ch_v, [rows_s[s], t_vec])
                rw = plsc.load_gather(rw_v, [rows_s[s], t_vec])
                kpos = ch * _NT + rows_s[s]
                chosen = plsc.load_gather(vtab, [kpos])
                pe = rw - chosen
                pe = jnp.where(rw != rw, 0.0, pe)
                coef = jnp.where(pe >= 0, ap, am)
                plsc.store_scatter(vtab, [kpos], chosen + coef * pe)
            return carry

        lax.fori_loop(0, _TCH, step, 0)
        for tl in range(_TCH):
            for kb in range(_K // 8):
                off = ((c * _TCH + tl) * tslab + kb * 8 * (N // _NT) * _NT
                       + wid * _NT * 8)
                pltpu.async_copy(
                    stage_v.at[pl.ds(tl * _K * _NT + kb * 8 * _NT, 8 * _NT)],
                    out_hbm.at[pl.ds(off, 8 * _NT)], sem)

    def pair(p, carry):
        run_chunk(2 * p, stage_a, sem_a, p == 0)
        run_chunk(2 * p + 1, stage_b, sem_b, p == 0)
        return carry

    npairs = T // (2 * _TCH)
    lax.fori_loop(0, npairs, pair, 0)
    pltpu.make_async_copy(stage_a, out_hbm.at[pl.ds(0, slab)], sem_a).wait()
    pltpu.make_async_copy(stage_b, out_hbm.at[pl.ds(0, slab)], sem_b).wait()


def kernel(choices, rewards, alpha_plus, alpha_minus, initial_values):
    N, T = choices.shape
    iv = 100.0 * jnp.tanh(initial_values)
    ap = jax.nn.sigmoid(alpha_plus)
    am = jax.nn.sigmoid(alpha_minus)
    params = jnp.concatenate([
        jnp.full((_L,), iv, jnp.float32),
        jnp.full((_L,), ap, jnp.float32),
        jnp.full((_L,), am, jnp.float32),
    ])

    mesh = plsc.VectorSubcoreMesh(core_axis_name="c", subcore_axis_name="s")
    run = pl.kernel(
        functools.partial(_sc_body, N, T),
        out_type=jax.ShapeDtypeStruct((N * T * _K,), jnp.float32),
        mesh=mesh,
        scratch_types=[
            pltpu.VMEM((3 * _L,), jnp.float32),
            pltpu.VMEM((_NT, T + 1), jnp.int32),
            pltpu.VMEM((_NT, T + 1), jnp.float32),
            pltpu.VMEM((_TCH * _K * _NT,), jnp.float32),
            pltpu.VMEM((_TCH * _K * _NT,), jnp.float32),
            pltpu.VMEM((_K * _NT,), jnp.float32),
            pltpu.SemaphoreType.DMA,
            pltpu.SemaphoreType.DMA,
        ],
        compiler_params=pltpu.CompilerParams(
            use_tc_tiling_on_sc=False, needs_layout_passes=False),
    )
    out5d = run(params, choices, rewards).reshape(
        T, _K // 8, N // _NT, 8, _NT)
    return out5d.transpose((2, 4, 0, 1, 3)).reshape(N, T, _K)


# speedup vs baseline: 2.7314x; 2.7314x over previous
import functools

import jax
import jax.numpy as jnp
from jax import lax
from jax.experimental import pallas as pl
from jax.experimental.pallas import tpu as pltpu
from jax.experimental.pallas import tpu_sc as plsc

_K = 32
_L = 16
_NW = 32
_NT = 128


def _sc_body(N, T, params_hbm, ch_hbm, rw_hbm, out_hbm,
             params_v, ch_v, rw_v, vtab, sem):
    wid = lax.axis_index("s") * 2 + lax.axis_index("c")
    n0 = wid * _NT
    kbs = _K // 8
    piece = 8 * _NT
    tslab = kbs * (N // _NT) * piece

    pltpu.sync_copy(params_hbm, params_v)
    iv = params_v[pl.ds(0, _L)]
    ap = params_v[pl.ds(_L, _L)]
    am = params_v[pl.ds(2 * _L, _L)]
    iota = lax.iota(jnp.int32, _L)
    rows_s = [iota + s * _L for s in range(_NT // _L)]

    pltpu.sync_copy(ch_hbm.at[pl.ds(n0, _NT), :], ch_v.at[:, pl.ds(0, T)])
    pltpu.sync_copy(rw_hbm.at[pl.ds(n0, _NT), :], rw_v.at[:, pl.ds(0, T)])

    for k in range(_K):
        for j in range(_NT // _L):
            vtab[pl.ds(k * _NT + j * _L, _L)] = iv

    def step(t, carry):
        base = t * tslab + wid * piece
        for kb in range(kbs):
            pltpu.async_copy(
                vtab.at[pl.ds(kb * piece, piece)],
                out_hbm.at[pl.ds(base + kb * (N // _NT) * piece, piece)],
                sem)
        t_vec = jnp.full((_L,), t, jnp.int32)
        upds = []
        for s in range(_NT // _L):
            ch = plsc.load_gather(ch_v, [rows_s[s], t_vec])
            rw = plsc.load_gather(rw_v, [rows_s[s], t_vec])
            kpos = ch * _NT + rows_s[s]
            chosen = plsc.load_gather(vtab, [kpos])
            pe = rw - chosen
            pe = jnp.where(rw != rw, 0.0, pe)
            coef = jnp.where(pe >= 0, ap, am)
            upds.append((kpos, chosen + coef * pe))
        pltpu.make_async_copy(vtab, out_hbm.at[pl.ds(0, _K * _NT)],
                              sem).wait()
        for kpos, upd in upds:
            plsc.store_scatter(vtab, [kpos], upd)
        return carry

    lax.fori_loop(0, T, step, 0)


def kernel(choices, rewards, alpha_plus, alpha_minus, initial_values):
    N, T = choices.shape
    iv = 100.0 * jnp.tanh(initial_values)
    ap = jax.nn.sigmoid(alpha_plus)
    am = jax.nn.sigmoid(alpha_minus)
    params = jnp.concatenate([
        jnp.full((_L,), iv, jnp.float32),
        jnp.full((_L,), ap, jnp.float32),
        jnp.full((_L,), am, jnp.float32),
    ])

    mesh = plsc.VectorSubcoreMesh(core_axis_name="c", subcore_axis_name="s")
    run = pl.kernel(
        functools.partial(_sc_body, N, T),
        out_type=jax.ShapeDtypeStruct((N * T * _K,), jnp.float32),
        mesh=mesh,
        scratch_types=[
            pltpu.VMEM((3 * _L,), jnp.float32),
            pltpu.VMEM((_NT, T + 1), jnp.int32),
            pltpu.VMEM((_NT, T + 1), jnp.float32),
            pltpu.VMEM((_K * _NT,), jnp.float32),
            pltpu.SemaphoreType.DMA,
        ],
        compiler_params=pltpu.CompilerParams(
            use_tc_tiling_on_sc=False, needs_layout_passes=False),
    )
    out5d = run(params, choices, rewards).reshape(
        T, _K // 8, N // _NT, 8, _NT)
    return out5d.transpose((2, 4, 0, 1, 3)).reshape(N, T, _K)
